# baseline (speedup 1.0000x reference)
import functools

import jax
import jax.numpy as jnp
from jax import lax
from jax.experimental import pallas as pl
from jax.experimental.pallas import tpu as pltpu
from jax.experimental.pallas import tpu_sc as plsc

N = 10000
E = 320000
D = 128
H = 128
WIN = 8

NC = 2
NS = 16
NW = NC * NS
EW = E // NW
C = 80
NCHUNK = EW // C
ROT = 1

RNG = 312
ACCR = 336
DUMP = 328
CAP = 12816
CB = 4000
NCB = E // CB


def _node_encode_body(cur_ref, gc_ref, adj_ref, w_ref, b_ref, nw_ref, nb_ref,
                      aw_ref, nf_ref, proj_ref, ts_ref):
    t = cur_ref[0]
    adj = adj_ref[...]
    dt = jnp.abs(t - adj)
    wv = w_ref[...]
    bv = b_ref[...]
    per = jnp.cos(dt * wv + bv)
    lin = w_ref[0, 0] * dt + b_ref[0, 0]
    lane = lax.broadcasted_iota(jnp.int32, per.shape, 1)
    tv = jnp.where(lane == 0, lin, per)
    eff = gc_ref[...] * nw_ref[...] + nb_ref[...]
    nf0 = jnp.maximum(eff + tv, 0.0)
    nrm = jnp.sqrt(jnp.sum(nf0 * nf0, axis=1, keepdims=True))
    nf = nf0 / jnp.maximum(nrm, 1e-12)
    nf_ref[...] = nf
    proj_ref[...] = jnp.dot(nf, aw_ref[...], preferred_element_type=jnp.float32)
    ts_ref[...] = 1.0 / jnp.log(jnp.e + 2.0 * (t - adj))


def _node_encode(cur_time, dict_gc, adj_time, t2v_w, t2v_b, node_w, node_b, att_w):
    b = 1000
    grid = (N // b,)
    return pl.pallas_call(
        _node_encode_body,
        grid=grid,
        in_specs=[
            pl.BlockSpec(memory_space=pltpu.SMEM),
            pl.BlockSpec((b, 1), lambda i: (i, 0)),
            pl.BlockSpec((b, 1), lambda i: (i, 0)),
            pl.BlockSpec((1, D), lambda i: (0, 0)),
            pl.BlockSpec((1, D), lambda i: (0, 0)),
            pl.BlockSpec((1, D), lambda i: (0, 0)),
            pl.BlockSpec((1, D), lambda i: (0, 0)),
            pl.BlockSpec((D, D), lambda i: (0, 0)),
        ],
        out_specs=[
            pl.BlockSpec((b, D), lambda i: (i, 0)),
            pl.BlockSpec((b, D), lambda i: (i, 0)),
            pl.BlockSpec((b, 1), lambda i: (i, 0)),
        ],
        out_shape=[
            jax.ShapeDtypeStruct((N, D), jnp.float32),
            jax.ShapeDtypeStruct((N, D), jnp.float32),
            jax.ShapeDtypeStruct((N, 1), jnp.float32),
        ],
    )(cur_time, dict_gc.reshape(N, 1), adj_time.reshape(N, 1),
      t2v_w.reshape(1, D), t2v_b.reshape(1, D), node_w.reshape(1, D),
      node_b.reshape(1, D), att_w)


def _hist_body(cur_ref, ht_ref, hf_ref, fh_ref):
    t = cur_ref[0]
    w = 1.0 / (1.0 + 2.0 * (t - ht_ref[...]))
    fh_ref[...] = jnp.sum(w[..., None] * hf_ref[...], axis=1)


def _hist_reduce(cur_time, hist_time, hist_feat):
    b = 400
    grid = (N // b,)
    return pl.pallas_call(
        _hist_body,
        grid=grid,
        in_specs=[
            pl.BlockSpec(memory_space=pltpu.SMEM),
            pl.BlockSpec((b, WIN), lambda i: (i, 0)),
            pl.BlockSpec((b, WIN, 2 * D), lambda i: (i, 0, 0)),
        ],
        out_specs=pl.BlockSpec((b, 2 * D), lambda i: (i, 0)),
        out_shape=jax.ShapeDtypeStruct((N, 2 * D), jnp.float32),
    )(cur_time, hist_time, hist_feat)


def _sc_score_body(nf_hbm, pj_hbm, ts_hbm, src_hbm, dst_hbm, sco_hbm,
                   srcv, dstv, nfr, pjr, scob, tst, sem1, sem2):
    c = lax.axis_index("c")
    s = lax.axis_index("s")
    wid = c * NS + s
    pltpu.sync_copy(ts_hbm, tst)
    lanes = lax.iota(jnp.int32, 16)

    def chunk(g, carry):
        base = wid * EW + g * C
        pltpu.sync_copy(src_hbm.at[pl.ds(base, C)], srcv)
        pltpu.sync_copy(dst_hbm.at[pl.ds(base, C)], dstv)
        cp1 = pltpu.async_copy(nf_hbm.at[srcv], nfr, sem1)
        cp2 = pltpu.async_copy(pj_hbm.at[dstv], pjr, sem2)
        cp1.wait()
        cp2.wait()
        for i in range(C // 16):
            row16 = lanes + (i * 16)
            src16 = srcv[pl.ds(i * 16, 16)]
            ts16 = plsc.load_gather(tst, [src16])
            rot = lanes * ROT

            def dbody(dd, a):
                col = jnp.bitwise_and(rot + dd, D - 1)
                x = plsc.load_gather(nfr, [row16, col])
                y = plsc.load_gather(pjr, [row16, col])
                return a + x * y

            att = plsc.parallel_loop(
                0, D, unroll=8, carry=jnp.zeros((16,), jnp.float32))(dbody)
            sc = ts16 + att
            scob[pl.ds(i * 16, 16)] = jnp.where(sc > 0.0, sc, 0.01 * sc)
        pltpu.sync_copy(scob, sco_hbm.at[pl.ds(base, C)])
        return carry

    lax.fori_loop(0, NCHUNK, chunk, 0)


def _sc_score(nf, proj, ts, src, dst):
    mesh = plsc.VectorSubcoreMesh(core_axis_name="c", subcore_axis_name="s")
    fn = pl.kernel(
        _sc_score_body,
        out_type=jax.ShapeDtypeStruct((E,), jnp.float32),
        mesh=mesh,
        scratch_types=[
            pltpu.VMEM((C,), jnp.int32),
            pltpu.VMEM((C,), jnp.int32),
            pltpu.VMEM((C, D), jnp.float32),
            pltpu.VMEM((C, D), jnp.float32),
            pltpu.VMEM((C,), jnp.float32),
            pltpu.VMEM((N,), jnp.float32),
            pltpu.SemaphoreType.DMA,
            pltpu.SemaphoreType.DMA,
        ],
        compiler_params=pltpu.CompilerParams(needs_layout_passes=False),
    )
    return fn(nf, proj, ts, src, dst)


def _sc_reduce_body(nf_hbm, src_hbm, dst_hbm, sco_hbm, zero_hbm, out_hbm,
                    dstb, srcb, scob, lsrc, ldst, lsco, nfr, acc, sem1):
    c = lax.axis_index("c")
    s = lax.axis_index("s")
    wid = c * NS + s
    lo = wid * RNG
    hi = lo + jnp.where(wid == NW - 1, ACCR - 8, RNG)
    lanes = lax.iota(jnp.int32, 16)

    pltpu.sync_copy(zero_hbm.at[pl.ds(0, ACCR)], acc)

    def pad_body(q):
        off = q * 16
        lsrc[pl.ds(off, 16)] = jnp.zeros((16,), jnp.int32)
        ldst[pl.ds(off, 16)] = jnp.full((16,), DUMP, jnp.int32)
        lsco[pl.ds(off, 16)] = jnp.zeros((16,), jnp.float32)

    plsc.parallel_loop(0, CAP // 16, unroll=4)(pad_body)

    def scan_chunk(gg, cnt):
        base = gg * CB
        pltpu.sync_copy(dst_hbm.at[pl.ds(base, CB)], dstb)
        pltpu.sync_copy(src_hbm.at[pl.ds(base, CB)], srcb)
        pltpu.sync_copy(sco_hbm.at[pl.ds(base, CB)], scob)

        def scan_grp(q, cnt2):
            off = q * 16
            d16 = dstb[pl.ds(off, 16)]
            m = jnp.logical_and(d16 >= lo, d16 < hi)
            plsc.store_compressed(lsrc.at[pl.ds(cnt2, 16)],
                                  srcb[pl.ds(off, 16)], mask=m)
            plsc.store_compressed(ldst.at[pl.ds(cnt2, 16)], d16 - lo, mask=m)
            plsc.store_compressed(lsco.at[pl.ds(cnt2, 16)],
                                  scob[pl.ds(off, 16)], mask=m)
            return cnt2 + jnp.sum(m.astype(jnp.int32))

        return lax.fori_loop(0, CB // 16, scan_grp, cnt)

    cnt = lax.fori_loop(0, NCB, scan_chunk, jnp.int32(0))

    ntrip = (cnt + (C - 1)) // C

    def acc_chunk(gg, carry):
        base = pl.multiple_of(gg * C, 8)
        pltpu.async_copy(nf_hbm.at[lsrc.at[pl.ds(base, C)]], nfr, sem1).wait()

        def grp_body(j):
            dv = ldst[pl.ds(base + j * 16, 16)]
            sv = lsco[pl.ds(base + j * 16, 16)]
            for e in range(16):
                dspl = jnp.full((16,), dv[e], jnp.int32)
                espl = jnp.full((16,), j * 16 + e, jnp.int32)
                sco = sv[e]
                for k in range(D // 16):
                    col = lanes + (k * 16)
                    x = plsc.load_gather(nfr, [espl, col])
                    plsc.addupdate_scatter(acc, [dspl, col], x * sco)

        plsc.parallel_loop(0, C // 16)(grp_body)
        return carry

    lax.fori_loop(0, ntrip, acc_chunk, 0)

    obase = pl.multiple_of(wid * RNG, 8)
    pltpu.sync_copy(acc.at[pl.ds(0, RNG)], out_hbm.at[pl.ds(obase, RNG)])

    @pl.when(wid == NW - 1)
    def _():
        pltpu.sync_copy(acc.at[pl.ds(RNG, 16)],
                        out_hbm.at[pl.ds(NW * RNG, 16)])


def _sc_reduce(nf, src, dst, score):
    mesh = plsc.VectorSubcoreMesh(core_axis_name="c", subcore_axis_name="s")
    zero = jnp.zeros((N, D), jnp.float32)
    fn = pl.kernel(
        _sc_reduce_body,
        out_type=jax.ShapeDtypeStruct((N, D), jnp.float32),
        mesh=mesh,
        scratch_types=[
            pltpu.VMEM((CB,), jnp.int32),
            pltpu.VMEM((CB,), jnp.int32),
            pltpu.VMEM((CB,), jnp.float32),
            pltpu.VMEM((CAP,), jnp.int32),
            pltpu.VMEM((CAP,), jnp.int32),
            pltpu.VMEM((CAP,), jnp.float32),
            pltpu.VMEM((C, D), jnp.float32),
            pltpu.VMEM((ACCR, D), jnp.float32),
            pltpu.SemaphoreType.DMA,
        ],
        compiler_params=pltpu.CompilerParams(needs_layout_passes=False),
    )
    return fn(nf, src, dst, score, zero)


def _combine_body(fh_ref, nf_ref, ng_ref, w1_ref, w2_ref, w3_ref, o_ref):
    dn = (((1,), (1,)), ((), ()))
    acc = lax.dot_general(fh_ref[...], w1_ref[...], dn,
                          preferred_element_type=jnp.float32)
    acc += lax.dot_general(nf_ref[...], w2_ref[...], dn,
                           preferred_element_type=jnp.float32)
    acc += lax.dot_general(ng_ref[...], w3_ref[...], dn,
                           preferred_element_type=jnp.float32)
    o_ref[...] = jnp.maximum(acc, 0.0)


def _combine(fh, nf, neigh, weight):
    b = 1000
    grid = (N // b,)
    w1 = weight[:, : 2 * D]
    w2 = weight[:, 2 * D: 3 * D]
    w3 = weight[:, 3 * D:]
    return pl.pallas_call(
        _combine_body,
        grid=grid,
        in_specs=[
            pl.BlockSpec((b, 2 * D), lambda i: (i, 0)),
            pl.BlockSpec((b, D), lambda i: (i, 0)),
            pl.BlockSpec((b, D), lambda i: (i, 0)),
            pl.BlockSpec((H, 2 * D), lambda i: (0, 0)),
            pl.BlockSpec((H, D), lambda i: (0, 0)),
            pl.BlockSpec((H, D), lambda i: (0, 0)),
        ],
        out_specs=pl.BlockSpec((b, H), lambda i: (i, 0)),
        out_shape=jax.ShapeDtypeStruct((N, H), jnp.float32),
    )(fh, nf, neigh, w1, w2, w3)


def kernel(edge_index, dict_gc, adj_time, cur_time, hist_feat, hist_time,
           t2v_w, t2v_b, node_w, node_b, att_w, weight):
    src = edge_index[0].astype(jnp.int32)
    dst = edge_index[1].astype(jnp.int32)
    nf, proj, ts2 = _node_encode(cur_time, dict_gc, adj_time, t2v_w, t2v_b,
                                 node_w[:, 0], node_b, att_w)
    score = _sc_score(nf, proj, ts2.reshape(N), src, dst)
    neigh = _sc_reduce(nf, src, dst, score)
    fh = _hist_reduce(cur_time, hist_time, hist_feat)
    return _combine(fh, nf, neigh, weight)


# speedup vs baseline: 3.5525x; 3.5525x over previous
import functools

import jax
import jax.numpy as jnp
from jax import lax
from jax.experimental import pallas as pl
from jax.experimental.pallas import tpu as pltpu
from jax.experimental.pallas import tpu_sc as plsc

N = 10000
E = 320000
D = 128
H = 128
WIN = 8

NC = 2
NS = 16
NW = NC * NS
EW = E // NW
C = 64
NCHUNK = EW // C
REM = EW - NCHUNK * C
RPT = 624
ROT = 1


def _node_encode_body(cur_ref, gc_ref, adj_ref, w_ref, b_ref, nw_ref, nb_ref,
                      aw_ref, nf_ref, proj_ref, ts_ref):
    t = cur_ref[0]
    adj = adj_ref[...]
    dt = jnp.abs(t - adj)
    wv = w_ref[...]
    bv = b_ref[...]
    per = jnp.cos(dt * wv + bv)
    lin = w_ref[0, 0] * dt + b_ref[0, 0]
    lane = lax.broadcasted_iota(jnp.int32, per.shape, 1)
    tv = jnp.where(lane == 0, lin, per)
    eff = gc_ref[...] * nw_ref[...] + nb_ref[...]
    nf0 = jnp.maximum(eff + tv, 0.0)
    nrm = jnp.sqrt(jnp.sum(nf0 * nf0, axis=1, keepdims=True))
    nf = nf0 / jnp.maximum(nrm, 1e-12)
    nf_ref[...] = nf
    proj_ref[...] = jnp.dot(nf, aw_ref[...], preferred_element_type=jnp.float32)
    ts_ref[...] = 1.0 / jnp.log(jnp.e + 2.0 * (t - adj))


def _node_encode(cur_time, dict_gc, adj_time, t2v_w, t2v_b, node_w, node_b, att_w):
    b = 1000
    grid = (N // b,)
    return pl.pallas_call(
        _node_encode_body,
        grid=grid,
        in_specs=[
            pl.BlockSpec(memory_space=pltpu.SMEM),
            pl.BlockSpec((b, 1), lambda i: (i, 0)),
            pl.BlockSpec((b, 1), lambda i: (i, 0)),
            pl.BlockSpec((1, D), lambda i: (0, 0)),
            pl.BlockSpec((1, D), lambda i: (0, 0)),
            pl.BlockSpec((1, D), lambda i: (0, 0)),
            pl.BlockSpec((1, D), lambda i: (0, 0)),
            pl.BlockSpec((D, D), lambda i: (0, 0)),
        ],
        out_specs=[
            pl.BlockSpec((b, D), lambda i: (i, 0)),
            pl.BlockSpec((b, D), lambda i: (i, 0)),
            pl.BlockSpec((b, 1), lambda i: (i, 0)),
        ],
        out_shape=[
            jax.ShapeDtypeStruct((N, D), jnp.float32),
            jax.ShapeDtypeStruct((N, D), jnp.float32),
            jax.ShapeDtypeStruct((N, 1), jnp.float32),
        ],
    )(cur_time, dict_gc.reshape(N, 1), adj_time.reshape(N, 1),
      t2v_w.reshape(1, D), t2v_b.reshape(1, D), node_w.reshape(1, D),
      node_b.reshape(1, D), att_w)


def _hist_body(cur_ref, ht_ref, hf_ref, fh_ref):
    t = cur_ref[0]
    w = 1.0 / (1.0 + 2.0 * (t - ht_ref[...]))
    fh_ref[...] = jnp.sum(w[..., None] * hf_ref[...], axis=1)


def _hist_reduce(cur_time, hist_time, hist_feat):
    b = 400
    grid = (N // b,)
    return pl.pallas_call(
        _hist_body,
        grid=grid,
        in_specs=[
            pl.BlockSpec(memory_space=pltpu.SMEM),
            pl.BlockSpec((b, WIN), lambda i: (i, 0)),
            pl.BlockSpec((b, WIN, 2 * D), lambda i: (i, 0, 0)),
        ],
        out_specs=pl.BlockSpec((b, 2 * D), lambda i: (i, 0)),
        out_shape=jax.ShapeDtypeStruct((N, 2 * D), jnp.float32),
    )(cur_time, hist_time, hist_feat)


def _sc_edge_body(nf_hbm, pj_hbm, ts_hbm, sd_hbm, src_hbm, dst_hbm, zero_hbm,
                  out_hbm, sdv0, sdv1, nfr0, pjr0, nfr1, pjr1,
                  tsr0, tsr1, msg, dstm, srcr, dstr, acc, sem0, sem1, semsc):
    c = lax.axis_index("c")
    s = lax.axis_index("s")
    wid = c * NS + s

    rbase = pl.multiple_of(s * RPT, 8)
    pltpu.sync_copy(zero_hbm.at[pl.ds(rbase, RPT)], acc.at[pl.ds(rbase, RPT)])

    @pl.when(s == NS - 1)
    def _():
        pltpu.sync_copy(zero_hbm.at[pl.ds(NS * RPT, N - NS * RPT)],
                        acc.at[pl.ds(NS * RPT, N - NS * RPT)])

    plsc.subcore_barrier()

    lanes = lax.iota(jnp.int32, 16)
    rot = lanes * ROT
    bufs = ((sdv0, nfr0, pjr0, tsr0, sem0),
            (sdv1, nfr1, pjr1, tsr1, sem1))

    def issue(k, g):
        sdv, nfr, pjr, tsr, sem = bufs[k]
        pltpu.sync_copy(sd_hbm.at[wid, g], sdv)
        pltpu.async_copy(nf_hbm.at[sdv.at[0]], nfr, sem)
        pltpu.async_copy(pj_hbm.at[sdv.at[1]], pjr, sem)
        pltpu.async_copy(ts_hbm.at[sdv.at[0]], tsr, sem)

    def drain(k):
        sdv, nfr, pjr, tsr, sem = bufs[k]
        pltpu.make_async_copy(nf_hbm.at[sdv.at[0]], nfr, sem).wait()
        pltpu.make_async_copy(pj_hbm.at[sdv.at[1]], pjr, sem).wait()
        pltpu.make_async_copy(ts_hbm.at[sdv.at[0]], tsr, sem).wait()

    def att_all(k):
        _, nfr, pjr, tsr, _ = bufs[k]
        scores = []
        for i in range(C // 16):
            row16 = lanes + (i * 16)

            def dbody(dd, a):
                col = jnp.bitwise_and(rot + dd, D - 1)
                x = plsc.load_gather(nfr, [row16, col])
                y = plsc.load_gather(pjr, [row16, col])
                return a + x * y

            att = plsc.parallel_loop(
                0, D, unroll=8, carry=jnp.zeros((16,), jnp.float32))(dbody)
            sc = tsr[pl.ds(i * 16, 16)] + att
            scores.append(jnp.where(sc > 0.0, sc, 0.01 * sc))
        return scores

    def msg_all(k, scores):
        sdv, nfr, pjr, tsr, _ = bufs[k]
        for i in range(C // 16):
            row16 = lanes + (i * 16)
            score = scores[i]
            dstm[pl.ds(i * 16, 16)] = sdv[1, pl.ds(i * 16, 16)]

            def mbody(dd):
                col = jnp.bitwise_and(rot + dd, D - 1)
                x = plsc.load_gather(nfr, [row16, col])
                plsc.store_scatter(msg, [row16, col], x * score)

            plsc.parallel_loop(0, D, unroll=8)(mbody)

    def scatter_issue():
        pltpu.async_copy(msg, acc.at[dstm], sem=semsc, add=True)

    def scatter_drain():
        pltpu.make_async_copy(msg, acc.at[dstm], semsc).wait()

    issue(0, 0)

    @pl.loop(0, NCHUNK, step=2)
    def _(g):
        issue(1, g + 1)
        drain(0)
        scores = att_all(0)

        @pl.when(g > 0)
        def _():
            scatter_drain()

        msg_all(0, scores)
        scatter_issue()

        @pl.when(g + 2 < NCHUNK)
        def _():
            issue(0, g + 2)

        drain(1)
        scores = att_all(1)
        scatter_drain()
        msg_all(1, scores)
        scatter_issue()

    scatter_drain()

    rembase = wid * EW + NCHUNK * C
    pltpu.sync_copy(src_hbm.at[pl.ds(rembase, REM)], srcr)
    pltpu.sync_copy(dst_hbm.at[pl.ds(rembase, REM)], dstr)
    pltpu.async_copy(nf_hbm.at[srcr], nfr0.at[pl.ds(0, REM)], sem0).wait()
    pltpu.async_copy(pj_hbm.at[dstr], pjr0.at[pl.ds(0, REM)], sem0).wait()
    pltpu.async_copy(ts_hbm.at[srcr], tsr0.at[pl.ds(0, REM)], sem0).wait()
    row16 = lanes

    def dbody_r(dd, a):
        col = jnp.bitwise_and(rot + dd, D - 1)
        x = plsc.load_gather(nfr0, [row16, col])
        y = plsc.load_gather(pjr0, [row16, col])
        return a + x * y

    att_r = plsc.parallel_loop(
        0, D, unroll=8, carry=jnp.zeros((16,), jnp.float32))(dbody_r)
    sc_r = tsr0[pl.ds(0, 16)] + att_r
    score_r = jnp.where(sc_r > 0.0, sc_r, 0.01 * sc_r)

    def mbody_r(dd):
        col = jnp.bitwise_and(rot + dd, D - 1)
        x = plsc.load_gather(nfr0, [row16, col])
        plsc.store_scatter(msg, [row16, col], x * score_r)

    plsc.parallel_loop(0, D, unroll=8)(mbody_r)
    pltpu.sync_copy(msg.at[pl.ds(0, REM)], acc.at[dstr], add=True)

    plsc.subcore_barrier()
    obase = pl.multiple_of(c * N + s * RPT, 8)
    pltpu.sync_copy(acc.at[pl.ds(rbase, RPT)], out_hbm.at[pl.ds(obase, RPT)])

    @pl.when(s == NS - 1)
    def _():
        pltpu.sync_copy(acc.at[pl.ds(NS * RPT, N - NS * RPT)],
                        out_hbm.at[pl.ds(c * N + NS * RPT, N - NS * RPT)])


def _sc_edge(nf, proj, ts, src, dst):
    mesh = plsc.VectorSubcoreMesh(core_axis_name="c", subcore_axis_name="s")
    zero = jnp.zeros((N, D), jnp.float32)
    nring = NCHUNK * C
    sd = jnp.stack([src.reshape(NW, EW)[:, :nring].reshape(NW, NCHUNK, C),
                    dst.reshape(NW, EW)[:, :nring].reshape(NW, NCHUNK, C)],
                   axis=2)
    fn = pl.kernel(
        _sc_edge_body,
        out_type=jax.ShapeDtypeStruct((NC * N, D), jnp.float32),
        mesh=mesh,
        scratch_types=[
            pltpu.VMEM((2, C), jnp.int32),
            pltpu.VMEM((2, C), jnp.int32),
            pltpu.VMEM((C, D), jnp.float32),
            pltpu.VMEM((C, D), jnp.float32),
            pltpu.VMEM((C, D), jnp.float32),
            pltpu.VMEM((C, D), jnp.float32),
            pltpu.VMEM((C,), jnp.float32),
            pltpu.VMEM((C,), jnp.float32),
            pltpu.VMEM((C, D), jnp.float32),
            pltpu.VMEM((C,), jnp.int32),
            pltpu.VMEM((16,), jnp.int32),
            pltpu.VMEM((16,), jnp.int32),
            pltpu.VMEM_SHARED((N, D), jnp.float32),
            pltpu.SemaphoreType.DMA,
            pltpu.SemaphoreType.DMA,
            pltpu.SemaphoreType.DMA,
        ],
        compiler_params=pltpu.CompilerParams(needs_layout_passes=False),
    )
    return fn(nf, proj, ts, sd, src, dst, zero)


def _combine_body(fh_ref, nf_ref, a0_ref, a1_ref, w1_ref, w2_ref, w3_ref, o_ref):
    neigh = a0_ref[...] + a1_ref[...]
    dn = (((1,), (1,)), ((), ()))
    acc = lax.dot_general(fh_ref[...], w1_ref[...], dn,
                          preferred_element_type=jnp.float32)
    acc += lax.dot_general(nf_ref[...], w2_ref[...], dn,
                           preferred_element_type=jnp.float32)
    acc += lax.dot_general(neigh, w3_ref[...], dn,
                           preferred_element_type=jnp.float32)
    o_ref[...] = jnp.maximum(acc, 0.0)


def _combine(fh, nf, accs, weight):
    b = 1000
    grid = (N // b,)
    w1 = weight[:, : 2 * D]
    w2 = weight[:, 2 * D: 3 * D]
    w3 = weight[:, 3 * D:]
    nb = N // b
    return pl.pallas_call(
        _combine_body,
        grid=grid,
        in_specs=[
            pl.BlockSpec((b, 2 * D), lambda i: (i, 0)),
            pl.BlockSpec((b, D), lambda i: (i, 0)),
            pl.BlockSpec((b, D), lambda i: (i, 0)),
            pl.BlockSpec((b, D), lambda i: (i + nb, 0)),
            pl.BlockSpec((H, 2 * D), lambda i: (0, 0)),
            pl.BlockSpec((H, D), lambda i: (0, 0)),
            pl.BlockSpec((H, D), lambda i: (0, 0)),
        ],
        out_specs=pl.BlockSpec((b, H), lambda i: (i, 0)),
        out_shape=jax.ShapeDtypeStruct((N, H), jnp.float32),
    )(fh, nf, accs, accs, w1, w2, w3)


def kernel(edge_index, dict_gc, adj_time, cur_time, hist_feat, hist_time,
           t2v_w, t2v_b, node_w, node_b, att_w, weight):
    src = edge_index[0].astype(jnp.int32)
    dst = edge_index[1].astype(jnp.int32)
    nf, proj, ts2 = _node_encode(cur_time, dict_gc, adj_time, t2v_w, t2v_b,
                                 node_w[:, 0], node_b, att_w)
    accs = _sc_edge(nf, proj, ts2.reshape(N), src, dst)
    fh = _hist_reduce(cur_time, hist_time, hist_feat)
    return _combine(fh, nf, accs, weight)
